# baseline (speedup 1.0000x reference)
import functools

import jax
import jax.numpy as jnp
from jax import lax
from jax.experimental import pallas as pl
from jax.experimental.pallas import tpu as pltpu
from jax.experimental.pallas import tpu_sc as plsc

N = 32768
D = 32
K = 1024
BR = 8192
H = N // 2
GH = H // BR
NEL = N * D


def _body(is_tail, zb_ref, emb_ref, cnt0_ref, lsum0_ref, idx_ref, loss_ref,
          perp_ref, cnt_ref):
    i = pl.program_id(0)
    zb = zb_ref[...]
    emb = emb_ref[...]
    s1 = jnp.sum(zb * zb, axis=1, keepdims=True)
    m2 = jnp.dot(zb, emb * -2.0, preferred_element_type=jnp.float32)
    e2 = jnp.sum(emb * emb, axis=0, keepdims=True)
    d = (s1 + m2) + e2
    dmin = jnp.min(d, axis=1)
    mask = d == dmin[:, None]
    iota = lax.broadcasted_iota(jnp.int32, (BR, K), 1)
    idx = jnp.min(jnp.where(mask, iota, K), axis=1)
    idx_ref[0, 0, :] = idx.astype(jnp.int32)

    maskf = mask.astype(jnp.float32)
    ones_row = jnp.ones((1, BR), jnp.float32)
    cpart = jnp.dot(ones_row, maskf, preferred_element_type=jnp.float32)
    prev_c = jnp.where(i == 0, jnp.zeros_like(cpart), cnt_ref[...])
    counts = prev_c + cpart
    cnt_ref[...] = counts

    partial = jnp.sum(dmin)
    prev = jnp.where(i == 0, 0.0, loss_ref[0, 0])
    acc = prev + partial
    if not is_tail:
        loss_ref[0, 0] = acc

        @pl.when(i == GH - 1)
        def _emit_counts():
            perp_ref[...] = counts
    else:
        loss_ref[0, 0] = jnp.where(
            i == GH - 1, (lsum0_ref[0, 0] + acc) * (0.25 / NEL), acc)

        @pl.when(i == GH - 1)
        def _final():
            ctot = counts + cnt0_ref[...]
            p = ctot * (1.0 / N)
            ent = jnp.sum(p * jnp.log(p + 1e-10))
            perp_ref[0, 0] = jnp.exp(-ent)


def _head(flat, embeddings):
    return pl.pallas_call(
        functools.partial(_body, False),
        grid=(GH,),
        in_specs=[
            pl.BlockSpec((BR, D), lambda i: (i, 0)),
            pl.BlockSpec((D, K), lambda i: (0, 0)),
            pl.BlockSpec((1, K), lambda i: (0, 0)),
            pl.BlockSpec((1, 1), lambda i: (0, 0), memory_space=pltpu.SMEM),
        ],
        out_specs=[
            pl.BlockSpec((1, 1, BR), lambda i: (i, 0, 0)),
            pl.BlockSpec((1, 1), lambda i: (0, 0), memory_space=pltpu.SMEM),
            pl.BlockSpec((1, K), lambda i: (0, 0)),
        ],
        out_shape=[
            jax.ShapeDtypeStruct((GH, 1, BR), jnp.int32),
            jax.ShapeDtypeStruct((1, 1), jnp.float32),
            jax.ShapeDtypeStruct((1, K), jnp.float32),
        ],
        scratch_shapes=[pltpu.VMEM((1, K), jnp.float32)],
    )(flat, embeddings, jnp.zeros((1, K), jnp.float32),
      jnp.zeros((1, 1), jnp.float32))


def _tail(flat, embeddings, cnt0, lsum0):
    return pl.pallas_call(
        functools.partial(_body, True),
        grid=(GH,),
        in_specs=[
            pl.BlockSpec((BR, D), lambda i: (i, 0)),
            pl.BlockSpec((D, K), lambda i: (0, 0)),
            pl.BlockSpec((1, K), lambda i: (0, 0)),
            pl.BlockSpec((1, 1), lambda i: (0, 0), memory_space=pltpu.SMEM),
        ],
        out_specs=[
            pl.BlockSpec((1, 1, BR), lambda i: (i, 0, 0)),
            pl.BlockSpec((1, 1), lambda i: (0, 0), memory_space=pltpu.SMEM),
            pl.BlockSpec((1, 1), lambda i: (0, 0), memory_space=pltpu.SMEM),
        ],
        out_shape=[
            jax.ShapeDtypeStruct((GH, 1, BR), jnp.int32),
            jax.ShapeDtypeStruct((1, 1), jnp.float32),
            jax.ShapeDtypeStruct((1, 1), jnp.float32),
        ],
        scratch_shapes=[pltpu.VMEM((1, K), jnp.float32)],
    )(flat, embeddings, cnt0, lsum0)


_SC_MESH = plsc.VectorSubcoreMesh(core_axis_name="c", subcore_axis_name="s")
NW = 32
RPW = H // NW


@functools.partial(
    pl.kernel,
    mesh=_SC_MESH,
    compiler_params=pltpu.CompilerParams(use_tc_tiling_on_sc=False),
    out_type=jax.ShapeDtypeStruct((H, D), jnp.float32),
    scratch_types=[
        pltpu.VMEM((RPW,), jnp.int32),
        pltpu.VMEM((RPW, D), jnp.float32),
        pltpu.VMEM_SHARED((K, D), jnp.float32),
        pltpu.SemaphoreType.DMA,
    ],
)
def _gather_rows(tableT_hbm, idx_hbm, q_hbm, idx_v, rows_v, shared_tab, sem):
    c = lax.axis_index("c")
    s = lax.axis_index("s")
    wid = s * 2 + c
    base = wid * RPW
    @pl.when(s == 0)
    def _stage():
        pltpu.sync_copy(tableT_hbm, shared_tab)

    g = base // BR
    off = base % BR
    pltpu.sync_copy(idx_hbm.at[g, 0, pl.ds(off, RPW)], idx_v)
    plsc.subcore_barrier()
    pltpu.async_copy(shared_tab.at[idx_v], rows_v, sem).wait()
    pltpu.sync_copy(rows_v, q_hbm.at[pl.ds(base, RPW)])


def kernel(z, embeddings):
    flat = z.reshape(-1, D)
    tableT = embeddings.T
    idx0, lsum0, cnt0 = _head(flat[:H], embeddings)
    q0 = _gather_rows(tableT, idx0)
    idx1, loss_arr, perp_arr = _tail(flat[H:], embeddings, cnt0, lsum0)
    q1 = _gather_rows(tableT, idx1)
    q_flat = jnp.concatenate([q0, q1], axis=0)
    quantized_st = q_flat.reshape(z.shape)
    return quantized_st, loss_arr[0, 0], perp_arr[0, 0]


# speedup vs baseline: 1.3182x; 1.3182x over previous
import functools

import jax
import jax.numpy as jnp
from jax import lax
from jax.experimental import pallas as pl
from jax.experimental.pallas import tpu as pltpu
from jax.experimental.pallas import tpu_sc as plsc

N = 32768
D = 32
K = 1024
BR = 8192
G = N // BR
NEL = N * D


def _dist_argmin_body(zb_ref, emb_ref, idx_ref, loss_ref, perp_ref,
                      cnt_ref):
    i = pl.program_id(0)
    zb = zb_ref[...]
    emb = emb_ref[...]
    s1 = jnp.sum(zb * zb, axis=1, keepdims=True)
    m2 = jnp.dot(zb, emb * -2.0, preferred_element_type=jnp.float32)
    e2 = jnp.sum(emb * emb, axis=0, keepdims=True)
    d = (s1 + m2) + e2
    dmin = jnp.min(d, axis=1)
    mask = d == dmin[:, None]
    iota = lax.broadcasted_iota(jnp.int32, (BR, K), 1)
    idx = jnp.min(jnp.where(mask, iota, K), axis=1)
    idx_ref[0, 0, :] = idx.astype(jnp.int32)

    maskf = mask.astype(jnp.float32)
    ones_row = jnp.ones((1, BR), jnp.float32)
    cpart = jnp.dot(ones_row, maskf, preferred_element_type=jnp.float32)
    prev_c = jnp.where(i == 0, jnp.zeros_like(cpart), cnt_ref[...])
    counts = prev_c + cpart
    cnt_ref[...] = counts

    partial = jnp.sum(dmin)
    prev = jnp.where(i == 0, 0.0, loss_ref[0, 0])
    acc = prev + partial
    loss_ref[0, 0] = jnp.where(i == G - 1, acc * (0.25 / NEL), acc)

    @pl.when(i == G - 1)
    def _final():
        p = counts * (1.0 / N)
        ent = jnp.sum(p * jnp.log(p + 1e-10))
        perp_ref[0, 0] = jnp.exp(-ent)


def _dist_argmin(flat, embeddings):
    return pl.pallas_call(
        _dist_argmin_body,
        grid=(G,),
        in_specs=[
            pl.BlockSpec((BR, D), lambda i: (i, 0)),
            pl.BlockSpec((D, K), lambda i: (0, 0)),
        ],
        out_specs=[
            pl.BlockSpec((1, 1, BR), lambda i: (i, 0, 0)),
            pl.BlockSpec((1, 1), lambda i: (0, 0), memory_space=pltpu.SMEM),
            pl.BlockSpec((1, 1), lambda i: (0, 0), memory_space=pltpu.SMEM),
        ],
        out_shape=[
            jax.ShapeDtypeStruct((G, 1, BR), jnp.int32),
            jax.ShapeDtypeStruct((1, 1), jnp.float32),
            jax.ShapeDtypeStruct((1, 1), jnp.float32),
        ],
        scratch_shapes=[pltpu.VMEM((1, K), jnp.float32)],
    )(flat, embeddings)


_SC_MESH = plsc.VectorSubcoreMesh(core_axis_name="c", subcore_axis_name="s")
NW = 32
RPW = N // NW


@functools.partial(
    pl.kernel,
    mesh=_SC_MESH,
    compiler_params=pltpu.CompilerParams(use_tc_tiling_on_sc=False),
    out_type=jax.ShapeDtypeStruct((N, D), jnp.float32),
    scratch_types=[
        pltpu.VMEM((RPW,), jnp.int32),
        pltpu.VMEM((RPW, D), jnp.float32),
        pltpu.VMEM_SHARED((K, D), jnp.float32),
        pltpu.SemaphoreType.DMA,
    ],
)
def _gather_rows(tableT_hbm, idx_hbm, q_hbm, idx_v, rows_v, shared_tab, sem):
    c = lax.axis_index("c")
    s = lax.axis_index("s")
    wid = s * 2 + c
    base = wid * RPW
    @pl.when(s == 0)
    def _stage():
        pltpu.sync_copy(tableT_hbm, shared_tab)

    g = base // BR
    off = base % BR
    pltpu.sync_copy(idx_hbm.at[g, 0, pl.ds(off, RPW)], idx_v)
    plsc.subcore_barrier()
    pltpu.async_copy(shared_tab.at[idx_v], rows_v, sem).wait()
    pltpu.sync_copy(rows_v, q_hbm.at[pl.ds(base, RPW)])


def kernel(z, embeddings):
    flat = z.reshape(-1, D)
    idx_blocks, loss_arr, perp_arr = _dist_argmin(flat, embeddings)
    tableT = embeddings.T
    q_flat = _gather_rows(tableT, idx_blocks)
    quantized_st = q_flat.reshape(z.shape)
    return quantized_st, loss_arr[0, 0], perp_arr[0, 0]
